# baseline (speedup 1.0000x reference)
import functools

import jax
import jax.numpy as jnp
from jax import lax
from jax.experimental import pallas as pl
from jax.experimental.pallas import tpu as pltpu
from jax.experimental.pallas import tpu_sc as plsc

_NC, _NS = 2, 16
_NW = _NC * _NS


def _argmin_body(xyz1t_ref, xyz2t_ref, idx_ref):
    b = pl.program_id(0)
    x1 = xyz1t_ref[0].T
    x2 = xyz2t_ref[0]
    d = (x1[:, 0:1] - x2[0:1, :]) ** 2
    d = d + (x1[:, 1:2] - x2[1:2, :]) ** 2
    d = d + (x1[:, 2:3] - x2[2:3, :]) ** 2
    n2 = d.shape[1]
    dmin = jnp.min(d, axis=1, keepdims=True)
    jidx = lax.broadcasted_iota(jnp.int32, d.shape, 1)
    idx = jnp.min(jnp.where(d == dmin, jidx, n2), axis=1, keepdims=True)
    idx_ref[0, 0] = idx.T + b * n2


def _mlp_body(interp_ref, p1_ref, w1a_ref, w1b_ref, b1_ref, w2_ref, b2_ref,
              out_ref):
    f2 = w1a_ref.shape[0]
    h = interp_ref[:, :f2] @ w1a_ref[...] + p1_ref[...] @ w1b_ref[...] + b1_ref[...]
    h = jnp.where(h >= 0, h, 0.2 * h)
    o = h @ w2_ref[...] + b2_ref[...]
    out_ref[0] = jnp.where(o >= 0, o, 0.2 * o)


def _sc_gather(table, idx3):
    d = table.shape[1]
    nw, ch, lw = idx3.shape
    bn = nw * ch * lw
    rounds, cpr = 2, ch // 2
    rows_per_round = cpr * lw
    mesh = plsc.VectorSubcoreMesh(core_axis_name="c", subcore_axis_name="s",
                                  num_cores=_NC, num_subcores=_NS)

    @functools.partial(
        pl.kernel,
        out_type=jax.ShapeDtypeStruct((bn, d), jnp.float32),
        mesh=mesh,
        scratch_types=[
            pltpu.VMEM((ch, lw), jnp.int32),
            pltpu.VMEM((rows_per_round, d), jnp.float32),
            pltpu.SemaphoreType.DMA,
        ],
    )
    def gather_k(table_hbm, idx_hbm, out_hbm, idx_v, rows_v, sem):
        wid = lax.axis_index("s") * _NC + lax.axis_index("c")
        base = wid * (ch * lw)
        pltpu.sync_copy(idx_hbm.at[wid], idx_v)
        for r in range(rounds):
            copies = [
                pltpu.async_copy(
                    table_hbm.at[idx_v.at[r * cpr + j]],
                    rows_v.at[pl.ds(j * lw, lw)], sem)
                for j in range(cpr)
            ]
            for cp in copies:
                cp.wait()
            pltpu.sync_copy(
                rows_v,
                out_hbm.at[pl.ds(base + r * rows_per_round, rows_per_round)])

    return gather_k(table, idx3)


def kernel(xyz1, xyz2, points1, points2, W1, b1, W2, b2):
    B, N1, _ = xyz1.shape
    N2 = xyz2.shape[1]
    F1 = points1.shape[2]
    F2 = points2.shape[2]
    FO = W2.shape[1]
    xyz2t = jnp.swapaxes(xyz2, 1, 2)
    W1a = W1[:F2]
    W1b = W1[F2:]
    b1r = b1.reshape(1, -1)
    b2r = b2.reshape(1, -1)

    xyz1t = jnp.swapaxes(xyz1, 1, 2)
    RB = 2048
    idx3 = pl.pallas_call(
        _argmin_body,
        grid=(B, N1 // RB),
        in_specs=[
            pl.BlockSpec((1, 3, RB), lambda b, i: (b, 0, i)),
            pl.BlockSpec((1, 3, N2), lambda b, i: (b, 0, 0)),
        ],
        out_specs=pl.BlockSpec((1, 1, 1, RB), lambda b, i: (b, i, 0, 0)),
        out_shape=jax.ShapeDtypeStruct((B, N1 // RB, 1, RB), jnp.int32),
    )(xyz1t, xyz2t)

    table = jnp.pad(points2.reshape(B * N2, F2), ((0, 0), (0, 128 - F2)))
    idxw = idx3.reshape(_NW, (B * N1) // (_NW * 128), 128)
    interp = _sc_gather(table, idxw)

    RC = 4096
    return pl.pallas_call(
        _mlp_body,
        grid=(B * N1 // RC,),
        in_specs=[
            pl.BlockSpec((RC, 128), lambda i: (i, 0)),
            pl.BlockSpec((RC, F1), lambda i: (i, 0)),
            pl.BlockSpec(W1a.shape, lambda i: (0, 0)),
            pl.BlockSpec(W1b.shape, lambda i: (0, 0)),
            pl.BlockSpec(b1r.shape, lambda i: (0, 0)),
            pl.BlockSpec(W2.shape, lambda i: (0, 0)),
            pl.BlockSpec(b2r.shape, lambda i: (0, 0)),
        ],
        out_specs=pl.BlockSpec((RC // N1, N1, FO), lambda i: (i, 0, 0)),
        out_shape=jax.ShapeDtypeStruct((B, N1, FO), jnp.float32),
    )(interp, points1.reshape(B * N1, F1), W1a, W1b, b1r, W2, b2r)


# speedup vs baseline: 1.0238x; 1.0238x over previous
import functools

import jax
import jax.numpy as jnp
from jax import lax
from jax.experimental import pallas as pl
from jax.experimental.pallas import tpu as pltpu
from jax.experimental.pallas import tpu_sc as plsc

_NC, _NS = 2, 16
_NW = _NC * _NS


def _argmin_body(xyz1t_ref, xyz2t_ref, idx_ref):
    b = pl.program_id(0)
    x1 = xyz1t_ref[0].T
    x2 = xyz2t_ref[0]
    d = (x1[:, 0:1] - x2[0:1, :]) ** 2
    d = d + (x1[:, 1:2] - x2[1:2, :]) ** 2
    d = d + (x1[:, 2:3] - x2[2:3, :]) ** 2
    n2 = d.shape[1]
    dmin = jnp.min(d, axis=1, keepdims=True)
    jidx = lax.broadcasted_iota(jnp.int32, d.shape, 1)
    idx = jnp.min(jnp.where(d == dmin, jidx, n2), axis=1, keepdims=True)
    idx_ref[0, 0] = idx.T + b * n2


def _mlp_body(interp_ref, p1_ref, w1a_ref, w1b_ref, b1_ref, w2_ref, b2_ref,
              out_ref):
    f2 = w1a_ref.shape[0]
    h = interp_ref[:, :f2] @ w1a_ref[...] + p1_ref[...] @ w1b_ref[...] + b1_ref[...]
    h = jnp.where(h >= 0, h, 0.2 * h)
    o = h @ w2_ref[...] + b2_ref[...]
    out_ref[...] = jnp.where(o >= 0, o, 0.2 * o)


def _sc_gather(table, idx3):
    d = table.shape[1]
    nw, ch, lw = idx3.shape
    bn = nw * ch * lw
    rounds, cpr = 2, ch // 2
    rows_per_round = cpr * lw
    mesh = plsc.VectorSubcoreMesh(core_axis_name="c", subcore_axis_name="s",
                                  num_cores=_NC, num_subcores=_NS)

    @functools.partial(
        pl.kernel,
        out_type=jax.ShapeDtypeStruct((bn, d), jnp.float32),
        mesh=mesh,
        scratch_types=[
            pltpu.VMEM((ch, lw), jnp.int32),
            pltpu.VMEM((rows_per_round, d), jnp.float32),
            pltpu.SemaphoreType.DMA,
        ],
    )
    def gather_k(table_hbm, idx_hbm, out_hbm, idx_v, rows_v, sem):
        wid = lax.axis_index("s") * _NC + lax.axis_index("c")
        base = wid * (ch * lw)
        pltpu.sync_copy(idx_hbm.at[wid], idx_v)
        for r in range(rounds):
            copies = [
                pltpu.async_copy(
                    table_hbm.at[idx_v.at[r * cpr + j]],
                    rows_v.at[pl.ds(j * lw, lw)], sem)
                for j in range(cpr)
            ]
            for cp in copies:
                cp.wait()
            pltpu.sync_copy(
                rows_v,
                out_hbm.at[pl.ds(base + r * rows_per_round, rows_per_round)])

    return gather_k(table, idx3)


def kernel(xyz1, xyz2, points1, points2, W1, b1, W2, b2):
    B, N1, _ = xyz1.shape
    N2 = xyz2.shape[1]
    F1 = points1.shape[2]
    F2 = points2.shape[2]
    FO = W2.shape[1]
    xyz2t = jnp.swapaxes(xyz2, 1, 2)
    W1a = W1[:F2]
    W1b = W1[F2:]
    b1r = b1.reshape(1, -1)
    b2r = b2.reshape(1, -1)

    xyz1t = jnp.swapaxes(xyz1, 1, 2)
    RB = 2048
    idx3 = pl.pallas_call(
        _argmin_body,
        grid=(B, N1 // RB),
        in_specs=[
            pl.BlockSpec((1, 3, RB), lambda b, i: (b, 0, i)),
            pl.BlockSpec((1, 3, N2), lambda b, i: (b, 0, 0)),
        ],
        out_specs=pl.BlockSpec((1, 1, 1, RB), lambda b, i: (b, i, 0, 0)),
        out_shape=jax.ShapeDtypeStruct((B, N1 // RB, 1, RB), jnp.int32),
    )(xyz1t, xyz2t)

    table = jnp.pad(points2.reshape(B * N2, F2), ((0, 0), (0, 128 - F2)))
    idxw = idx3.reshape(_NW, (B * N1) // (_NW * 128), 128)
    interp = _sc_gather(table, idxw)

    RC = 4096
    out_flat = pl.pallas_call(
        _mlp_body,
        grid=(B * N1 // RC,),
        in_specs=[
            pl.BlockSpec((RC, 128), lambda i: (i, 0)),
            pl.BlockSpec((RC, F1), lambda i: (i, 0)),
            pl.BlockSpec(W1a.shape, lambda i: (0, 0)),
            pl.BlockSpec(W1b.shape, lambda i: (0, 0)),
            pl.BlockSpec(b1r.shape, lambda i: (0, 0)),
            pl.BlockSpec(W2.shape, lambda i: (0, 0)),
            pl.BlockSpec(b2r.shape, lambda i: (0, 0)),
        ],
        out_specs=pl.BlockSpec((RC, FO), lambda i: (i, 0)),
        out_shape=jax.ShapeDtypeStruct((B * N1, FO), jnp.float32),
    )(interp, points1.reshape(B * N1, F1), W1a, W1b, b1r, W2, b2r)

    return out_flat.reshape(B, N1, FO)
